# Initial kernel scaffold; baseline (speedup 1.0000x reference)
import jax
import jax.numpy as jnp
from jax.experimental import pallas as pl


def kernel(x, edge_index, edge_attr, params):
    raise NotImplementedError("write your pallas kernel here")


# speedup vs baseline: 2.1816x; 2.1816x over previous
import functools

import jax
import jax.numpy as jnp
from jax import lax
from jax.experimental import pallas as pl
from jax.experimental.pallas import tpu as pltpu
from jax.experimental.pallas import tpu_sc as plsc

F32 = jnp.float32

NC = 2
NS = 16
NW = NC * NS

SUB = 128
K_G = 25
K_S = 10

BN = 2000
BE = 4000


def _silu(x):
    return x * jax.nn.sigmoid(x)


def _embed_body(x_ref, w_ref, b_ref, h_ref, s8_ref):
    h = _silu(jnp.dot(x_ref[...], w_ref[...], preferred_element_type=F32)
              + b_ref[...])
    h_ref[...] = h
    s8_ref[...] = h[:, :8]


def _edge_body(si_ref, sj_ref, ea_ref, wi_ref, wj_ref, wd_ref, be1_ref,
               we2_ref, be2_ref, ma_ref, mb_ref, mc_ref, md_ref):
    si = si_ref[...]
    sj = sj_ref[...]
    ea = ea_ref[...]
    d1 = jnp.sum(si[:, 2:5] * sj[:, 2:5], axis=1, keepdims=True)
    d2 = jnp.sum(sj[:, 2:5] * ea[:, 0:3], axis=1, keepdims=True)
    rn = ea[:, 3:4]
    wd = wd_ref[...]
    pre = (jnp.dot(si, wi_ref[...], preferred_element_type=F32)
           + jnp.dot(sj, wj_ref[...], preferred_element_type=F32)
           + d1 * wd[0:1] + d2 * wd[1:2] + rn * wd[2:3] + be1_ref[...])
    h1 = _silu(pre)
    msg = _silu(jnp.dot(h1, we2_ref[...], preferred_element_type=F32)
                + be2_ref[...])
    ma_ref[...] = msg[:, 0:16]
    mb_ref[...] = msg[:, 16:32]
    mc_ref[...] = msg[:, 32:48]
    md_ref[...] = msg[:, 48:64]


def _node_body(h_ref, a0_ref, a1_ref, a2_ref, a3_ref, wa_ref, wagg_ref,
               b1_ref, w2_ref, b2_ref, hn_ref, s8_ref):
    h = h_ref[...]
    wagg = wagg_ref[...]
    pre = jnp.dot(h, wa_ref[...], preferred_element_type=F32) + b1_ref[...]
    for k, a_ref in enumerate((a0_ref, a1_ref, a2_ref, a3_ref)):
        pre = pre + jnp.dot(a_ref[0], wagg[16 * k:16 * (k + 1)],
                            preferred_element_type=F32)
    t = _silu(pre)
    hn = h + jnp.dot(t, w2_ref[...], preferred_element_type=F32) + b2_ref[...]
    hn_ref[...] = hn
    s8_ref[...] = hn[:, :8]


def _out_body(h_ref, w1_ref, b1_ref, w2_ref, b2_ref, o_ref):
    t = _silu(jnp.dot(h_ref[...], w1_ref[...], preferred_element_type=F32)
              + b1_ref[...])
    o_ref[...] = jnp.dot(t, w2_ref[...], preferred_element_type=F32) + b2_ref[...]


def _bcast(shape):
    return pl.BlockSpec(shape, lambda i: tuple(0 for _ in shape))


def _make_gather(n_nodes, n_edges):
    macro = K_G * SUB
    n_macro = n_edges // macro
    mesh = plsc.VectorSubcoreMesh(core_axis_name="c", subcore_axis_name="s")

    @functools.partial(
        pl.kernel,
        out_type=[jax.ShapeDtypeStruct((n_edges, 8), F32),
                  jax.ShapeDtypeStruct((n_edges, 8), F32)],
        mesh=mesh,
        scratch_types=[pltpu.VMEM((K_G, SUB), jnp.int32),
                       pltpu.VMEM((K_G, SUB), jnp.int32),
                       pltpu.VMEM((macro, 8), F32),
                       pltpu.VMEM((macro, 8), F32),
                       pltpu.SemaphoreType.DMA,
                       pltpu.SemaphoreType.DMA],
        compiler_params=pltpu.CompilerParams(use_tc_tiling_on_sc=False),
    )
    def gather(s8_hbm, dst3_hbm, src3_hbm, si_hbm, sj_hbm,
               idx_d, idx_s, rows_d, rows_s, sem_d, sem_s):
        wid = lax.axis_index("s") * NC + lax.axis_index("c")
        n_mine = (n_macro - 1 - wid) // NW + 1

        def macro_body(t, _):
            m = wid + t * NW
            e0 = m * macro
            pltpu.sync_copy(dst3_hbm.at[m], idx_d)
            pltpu.sync_copy(src3_hbm.at[m], idx_s)

            def fire(j, _):
                pltpu.make_async_copy(
                    s8_hbm.at[idx_d.at[j]],
                    rows_d.at[pl.ds(j * SUB, SUB)], sem_d).start()
                pltpu.make_async_copy(
                    s8_hbm.at[idx_s.at[j]],
                    rows_s.at[pl.ds(j * SUB, SUB)], sem_s).start()
                return 0

            lax.fori_loop(0, K_G, fire, 0)

            def drain(j, _):
                pltpu.make_async_copy(
                    s8_hbm.at[idx_d.at[j]],
                    rows_d.at[pl.ds(j * SUB, SUB)], sem_d).wait()
                pltpu.make_async_copy(
                    s8_hbm.at[idx_s.at[j]],
                    rows_s.at[pl.ds(j * SUB, SUB)], sem_s).wait()
                return 0

            lax.fori_loop(0, K_G, drain, 0)
            pltpu.sync_copy(rows_d, si_hbm.at[pl.ds(e0, macro)])
            pltpu.sync_copy(rows_s, sj_hbm.at[pl.ds(e0, macro)])
            return 0

        lax.fori_loop(0, n_mine, macro_body, 0)

    return gather


def _make_scatter(n_nodes, n_edges):
    macro = K_S * SUB
    n_macro = n_edges // macro
    stripe = -(-n_nodes // (8 * NS)) * 8
    n_pad = stripe * NS
    mesh = plsc.VectorSubcoreMesh(core_axis_name="c", subcore_axis_name="s")

    @functools.partial(
        pl.kernel,
        out_type=jax.ShapeDtypeStruct((4, n_nodes, 16), F32),
        mesh=mesh,
        scratch_types=[pltpu.VMEM((K_S, SUB), jnp.int32),
                       pltpu.VMEM((macro, 16), F32),
                       pltpu.VMEM_SHARED((n_pad, 16), F32),
                       pltpu.SemaphoreType.DMA],
        compiler_params=pltpu.CompilerParams(use_tc_tiling_on_sc=False),
    )
    def scatter(ma_hbm, mb_hbm, mc_hbm, md_hbm, dst3_hbm, zeros_hbm,
                aggr_hbm, idx_v, msg_v, acc, sem):
        c = lax.axis_index("c")
        s = lax.axis_index("s")
        n_mine = (n_macro - 1 - s) // NS + 1
        full = n_nodes // stripe
        rem = n_nodes - full * stripe
        halves = ((ma_hbm, mc_hbm), (mb_hbm, md_hbm))

        for p in range(2):
            pltpu.sync_copy(zeros_hbm, acc.at[pl.ds(s * stripe, stripe)])
            plsc.subcore_barrier()

            def macro_body(t, _):
                m = s + t * NS
                e0 = m * macro
                pltpu.sync_copy(dst3_hbm.at[m], idx_v)

                @pl.when(c == 0)
                def _():
                    pltpu.sync_copy(halves[p][0].at[pl.ds(e0, macro)], msg_v)

                @pl.when(c == 1)
                def _():
                    pltpu.sync_copy(halves[p][1].at[pl.ds(e0, macro)], msg_v)

                def fire(j, _):
                    pltpu.make_async_copy(
                        msg_v.at[pl.ds(j * SUB, SUB)],
                        acc.at[idx_v.at[j]], sem).start(add=True)
                    return 0

                lax.fori_loop(0, K_S, fire, 0)

                def drain(j, _):
                    pltpu.make_async_copy(
                        msg_v.at[pl.ds(j * SUB, SUB)],
                        acc.at[idx_v.at[j]], sem).wait()
                    return 0

                lax.fori_loop(0, K_S, drain, 0)
                return 0

            lax.fori_loop(0, n_mine, macro_body, 0)
            plsc.subcore_barrier()
            chunk = 2 * c + p

            @pl.when(s < full)
            def _():
                pltpu.sync_copy(acc.at[pl.ds(s * stripe, stripe)],
                                aggr_hbm.at[chunk, pl.ds(s * stripe, stripe)])

            if rem:
                @pl.when(s == full)
                def _():
                    pltpu.sync_copy(
                        acc.at[pl.ds(full * stripe, rem)],
                        aggr_hbm.at[chunk, pl.ds(full * stripe, rem)])

    return scatter


def kernel(x, edge_index, edge_attr, params):
    n_nodes, _ = x.shape
    n_edges = edge_index.shape[1]
    assert n_edges % (K_G * SUB) == 0 and n_edges % (K_S * SUB) == 0
    assert n_nodes % NS == 0 and n_nodes % BN == 0 and n_edges % BE == 0

    dst = edge_index[1]
    src = edge_index[0]
    dst3_g = dst.reshape(n_edges // (K_G * SUB), K_G, SUB)
    src3_g = src.reshape(n_edges // (K_G * SUB), K_G, SUB)
    dst3_s = dst.reshape(n_edges // (K_S * SUB), K_S, SUB)
    stripe = -(-n_nodes // (8 * NS)) * 8
    zeros_tile = jnp.zeros((stripe, 16), F32)

    gn = n_nodes // BN
    ge = n_edges // BE

    w0, b0 = params['emb']
    h, s8 = pl.pallas_call(
        _embed_body,
        grid=(gn,),
        in_specs=[pl.BlockSpec((BN, 5), lambda i: (i, 0)),
                  _bcast((5, 64)), _bcast((1, 64))],
        out_specs=[pl.BlockSpec((BN, 64), lambda i: (i, 0)),
                   pl.BlockSpec((BN, 8), lambda i: (i, 0))],
        out_shape=[jax.ShapeDtypeStruct((n_nodes, 64), F32),
                   jax.ShapeDtypeStruct((n_nodes, 8), F32)],
    )(x, w0, b0.reshape(1, 64))

    gather = _make_gather(n_nodes, n_edges)
    scatter = _make_scatter(n_nodes, n_edges)

    edge_call = pl.pallas_call(
        _edge_body,
        grid=(ge,),
        in_specs=[pl.BlockSpec((BE, 8), lambda i: (i, 0)),
                  pl.BlockSpec((BE, 8), lambda i: (i, 0)),
                  pl.BlockSpec((BE, 4), lambda i: (i, 0)),
                  _bcast((8, 64)), _bcast((8, 64)), _bcast((3, 64)),
                  _bcast((1, 64)), _bcast((64, 64)), _bcast((1, 64))],
        out_specs=[pl.BlockSpec((BE, 16), lambda i: (i, 0))] * 4,
        out_shape=[jax.ShapeDtypeStruct((n_edges, 16), F32)] * 4,
    )

    node_call = pl.pallas_call(
        _node_body,
        grid=(gn,),
        in_specs=[pl.BlockSpec((BN, 64), lambda i: (i, 0)),
                  pl.BlockSpec((1, BN, 16), lambda i: (0, i, 0)),
                  pl.BlockSpec((1, BN, 16), lambda i: (1, i, 0)),
                  pl.BlockSpec((1, BN, 16), lambda i: (2, i, 0)),
                  pl.BlockSpec((1, BN, 16), lambda i: (3, i, 0)),
                  _bcast((64, 64)), _bcast((64, 64)),
                  _bcast((1, 64)), _bcast((64, 64)), _bcast((1, 64))],
        out_specs=[pl.BlockSpec((BN, 64), lambda i: (i, 0)),
                   pl.BlockSpec((BN, 8), lambda i: (i, 0))],
        out_shape=[jax.ShapeDtypeStruct((n_nodes, 64), F32),
                   jax.ShapeDtypeStruct((n_nodes, 8), F32)],
    )

    for lp in params['layers']:
        we1, be1 = lp['edge1']
        we2, be2 = lp['edge2']
        wn1, bn1 = lp['node1']
        wn2, bn2 = lp['node2']
        z3 = jnp.zeros((3, 64), F32)
        wi = jnp.concatenate([we1[9:11], we1[0:3], z3], axis=0)
        wj = jnp.concatenate([we1[11:13], we1[3:6], z3], axis=0)
        wd = we1[6:9]

        si8, sj8 = gather(s8, dst3_g, src3_g)
        ma, mb, mc, md = edge_call(si8, sj8, edge_attr, wi, wj, wd,
                                   be1.reshape(1, 64), we2,
                                   be2.reshape(1, 64))
        aggr = scatter(ma, mb, mc, md, dst3_s, zeros_tile)
        h, s8 = node_call(h, aggr, aggr, aggr, aggr, wn1[:64], wn1[64:],
                          bn1.reshape(1, 64), wn2, bn2.reshape(1, 64))

    wo1, bo1 = params['out1']
    wo2, bo2 = params['out2']
    out = pl.pallas_call(
        _out_body,
        grid=(gn,),
        in_specs=[pl.BlockSpec((BN, 64), lambda i: (i, 0)),
                  _bcast((64, 64)), _bcast((1, 64)),
                  _bcast((64, 3)), _bcast((1, 3))],
        out_specs=pl.BlockSpec((BN, 3), lambda i: (i, 0)),
        out_shape=jax.ShapeDtypeStruct((n_nodes, 3), F32),
    )(h, wo1, bo1.reshape(1, 64), wo2, bo2.reshape(1, 3))
    return out


# speedup vs baseline: 2.2639x; 1.0377x over previous
import functools

import jax
import jax.numpy as jnp
from jax import lax
from jax.experimental import pallas as pl
from jax.experimental.pallas import tpu as pltpu
from jax.experimental.pallas import tpu_sc as plsc

F32 = jnp.float32

NC = 2
NS = 16
NW = NC * NS

SUB = 128
K_G = 25
K_S = 10

BN = 2000
BE = 3200


def _silu(x):
    return x * jax.nn.sigmoid(x)


def _embed_body(x_ref, w_ref, b_ref, h_ref, s8_ref):
    h = _silu(jnp.dot(x_ref[...], w_ref[...], preferred_element_type=F32)
              + b_ref[...])
    h_ref[...] = h
    s8_ref[...] = h[:, :8]


def _edge_body(si_ref, sj_ref, ea_ref, wi_ref, wj_ref, wp1_ref, wp2_ref,
               wea_ref, be1_ref, we2_ref, be2_ref,
               ma_ref, mb_ref, mc_ref, md_ref):
    si = si_ref[...]
    sj = sj_ref[...]
    ea = ea_ref[...]
    p1 = si * sj
    p2 = sj * ea
    pre = (jnp.dot(si, wi_ref[...], preferred_element_type=F32)
           + jnp.dot(sj, wj_ref[...], preferred_element_type=F32)
           + jnp.dot(p1, wp1_ref[...], preferred_element_type=F32)
           + jnp.dot(p2, wp2_ref[...], preferred_element_type=F32)
           + jnp.dot(ea, wea_ref[...], preferred_element_type=F32)
           + be1_ref[...])
    h1 = _silu(pre)
    msg = _silu(jnp.dot(h1, we2_ref[...], preferred_element_type=F32)
                + be2_ref[...])
    ma_ref[...] = msg[:, 0:16]
    mb_ref[...] = msg[:, 16:32]
    mc_ref[...] = msg[:, 32:48]
    md_ref[...] = msg[:, 48:64]


def _node_body(h_ref, a0_ref, a1_ref, a2_ref, a3_ref, wa_ref, wagg_ref,
               b1_ref, w2_ref, b2_ref, hn_ref, s8_ref):
    h = h_ref[...]
    wagg = wagg_ref[...]
    pre = jnp.dot(h, wa_ref[...], preferred_element_type=F32) + b1_ref[...]
    for k, a_ref in enumerate((a0_ref, a1_ref, a2_ref, a3_ref)):
        pre = pre + jnp.dot(a_ref[0], wagg[16 * k:16 * (k + 1)],
                            preferred_element_type=F32)
    t = _silu(pre)
    hn = h + jnp.dot(t, w2_ref[...], preferred_element_type=F32) + b2_ref[...]
    hn_ref[...] = hn
    s8_ref[...] = hn[:, :8]


def _out_body(h_ref, w1_ref, b1_ref, w2_ref, b2_ref, o_ref):
    t = _silu(jnp.dot(h_ref[...], w1_ref[...], preferred_element_type=F32)
              + b1_ref[...])
    o_ref[...] = jnp.dot(t, w2_ref[...], preferred_element_type=F32) + b2_ref[...]


def _bcast(shape):
    return pl.BlockSpec(shape, lambda i: tuple(0 for _ in shape))


def _make_gather(n_nodes, n_edges):
    macro = K_G * SUB
    n_macro = n_edges // macro
    mesh = plsc.VectorSubcoreMesh(core_axis_name="c", subcore_axis_name="s")

    @functools.partial(
        pl.kernel,
        out_type=[jax.ShapeDtypeStruct((n_edges, 8), F32),
                  jax.ShapeDtypeStruct((n_edges, 8), F32)],
        mesh=mesh,
        scratch_types=[pltpu.VMEM((K_G, SUB), jnp.int32),
                       pltpu.VMEM((K_G, SUB), jnp.int32),
                       pltpu.VMEM((macro, 8), F32),
                       pltpu.VMEM((macro, 8), F32),
                       pltpu.SemaphoreType.DMA,
                       pltpu.SemaphoreType.DMA],
        compiler_params=pltpu.CompilerParams(use_tc_tiling_on_sc=False),
    )
    def gather(s8_hbm, dst3_hbm, src3_hbm, si_hbm, sj_hbm,
               idx_d, idx_s, rows_d, rows_s, sem_d, sem_s):
        wid = lax.axis_index("s") * NC + lax.axis_index("c")
        n_mine = (n_macro - 1 - wid) // NW + 1

        def macro_body(t, _):
            m = wid + t * NW
            e0 = m * macro
            pltpu.sync_copy(dst3_hbm.at[m], idx_d)
            pltpu.sync_copy(src3_hbm.at[m], idx_s)

            def fire(j, _):
                pltpu.make_async_copy(
                    s8_hbm.at[idx_d.at[j]],
                    rows_d.at[pl.ds(j * SUB, SUB)], sem_d).start()
                pltpu.make_async_copy(
                    s8_hbm.at[idx_s.at[j]],
                    rows_s.at[pl.ds(j * SUB, SUB)], sem_s).start()
                return 0

            lax.fori_loop(0, K_G, fire, 0)

            def drain(j, _):
                pltpu.make_async_copy(
                    s8_hbm.at[idx_d.at[j]],
                    rows_d.at[pl.ds(j * SUB, SUB)], sem_d).wait()
                pltpu.make_async_copy(
                    s8_hbm.at[idx_s.at[j]],
                    rows_s.at[pl.ds(j * SUB, SUB)], sem_s).wait()
                return 0

            lax.fori_loop(0, K_G, drain, 0)
            pltpu.sync_copy(rows_d, si_hbm.at[pl.ds(e0, macro)])
            pltpu.sync_copy(rows_s, sj_hbm.at[pl.ds(e0, macro)])
            return 0

        lax.fori_loop(0, n_mine, macro_body, 0)

    return gather


def _make_scatter(n_nodes, n_edges):
    macro = K_S * SUB
    n_macro = n_edges // macro
    stripe = -(-n_nodes // (8 * NS)) * 8
    n_pad = stripe * NS
    mesh = plsc.VectorSubcoreMesh(core_axis_name="c", subcore_axis_name="s")

    @functools.partial(
        pl.kernel,
        out_type=jax.ShapeDtypeStruct((4, n_nodes, 16), F32),
        mesh=mesh,
        scratch_types=[pltpu.VMEM((K_S, SUB), jnp.int32),
                       pltpu.VMEM((macro, 16), F32),
                       pltpu.VMEM_SHARED((n_pad, 16), F32),
                       pltpu.SemaphoreType.DMA],
        compiler_params=pltpu.CompilerParams(use_tc_tiling_on_sc=False),
    )
    def scatter(ma_hbm, mb_hbm, mc_hbm, md_hbm, dst3_hbm, zeros_hbm,
                aggr_hbm, idx_v, msg_v, acc, sem):
        c = lax.axis_index("c")
        s = lax.axis_index("s")
        n_mine = (n_macro - 1 - s) // NS + 1
        full = n_nodes // stripe
        rem = n_nodes - full * stripe
        halves = ((ma_hbm, mc_hbm), (mb_hbm, md_hbm))

        for p in range(2):
            pltpu.sync_copy(zeros_hbm, acc.at[pl.ds(s * stripe, stripe)])
            plsc.subcore_barrier()

            def macro_body(t, _):
                m = s + t * NS
                e0 = m * macro
                pltpu.sync_copy(dst3_hbm.at[m], idx_v)

                @pl.when(c == 0)
                def _():
                    pltpu.sync_copy(halves[p][0].at[pl.ds(e0, macro)], msg_v)

                @pl.when(c == 1)
                def _():
                    pltpu.sync_copy(halves[p][1].at[pl.ds(e0, macro)], msg_v)

                def fire(j, _):
                    pltpu.make_async_copy(
                        msg_v.at[pl.ds(j * SUB, SUB)],
                        acc.at[idx_v.at[j]], sem).start(add=True)
                    return 0

                lax.fori_loop(0, K_S, fire, 0)

                def drain(j, _):
                    pltpu.make_async_copy(
                        msg_v.at[pl.ds(j * SUB, SUB)],
                        acc.at[idx_v.at[j]], sem).wait()
                    return 0

                lax.fori_loop(0, K_S, drain, 0)
                return 0

            lax.fori_loop(0, n_mine, macro_body, 0)
            plsc.subcore_barrier()
            chunk = 2 * c + p

            @pl.when(s < full)
            def _():
                pltpu.sync_copy(acc.at[pl.ds(s * stripe, stripe)],
                                aggr_hbm.at[chunk, pl.ds(s * stripe, stripe)])

            if rem:
                @pl.when(s == full)
                def _():
                    pltpu.sync_copy(
                        acc.at[pl.ds(full * stripe, rem)],
                        aggr_hbm.at[chunk, pl.ds(full * stripe, rem)])

    return scatter


def kernel(x, edge_index, edge_attr, params):
    n_nodes, _ = x.shape
    n_edges = edge_index.shape[1]
    assert n_edges % (K_G * SUB) == 0 and n_edges % (K_S * SUB) == 0
    assert n_nodes % NS == 0 and n_nodes % BN == 0 and n_edges % BE == 0

    dst = edge_index[1]
    src = edge_index[0]
    dst3_g = dst.reshape(n_edges // (K_G * SUB), K_G, SUB)
    src3_g = src.reshape(n_edges // (K_G * SUB), K_G, SUB)
    dst3_s = dst.reshape(n_edges // (K_S * SUB), K_S, SUB)
    stripe = -(-n_nodes // (8 * NS)) * 8
    zeros_tile = jnp.zeros((stripe, 16), F32)
    ez = jnp.zeros((n_edges, 2), F32)
    ea8 = jnp.concatenate([ez, edge_attr, ez], axis=1)

    gn = n_nodes // BN
    ge = n_edges // BE

    w0, b0 = params['emb']
    h, s8 = pl.pallas_call(
        _embed_body,
        grid=(gn,),
        in_specs=[pl.BlockSpec((BN, 5), lambda i: (i, 0)),
                  _bcast((5, 64)), _bcast((1, 64))],
        out_specs=[pl.BlockSpec((BN, 64), lambda i: (i, 0)),
                   pl.BlockSpec((BN, 8), lambda i: (i, 0))],
        out_shape=[jax.ShapeDtypeStruct((n_nodes, 64), F32),
                   jax.ShapeDtypeStruct((n_nodes, 8), F32)],
    )(x, w0, b0.reshape(1, 64))

    gather = _make_gather(n_nodes, n_edges)
    scatter = _make_scatter(n_nodes, n_edges)

    edge_call = pl.pallas_call(
        _edge_body,
        grid=(ge,),
        in_specs=[pl.BlockSpec((BE, 8), lambda i: (i, 0)),
                  pl.BlockSpec((BE, 8), lambda i: (i, 0)),
                  pl.BlockSpec((BE, 8), lambda i: (i, 0)),
                  _bcast((8, 64)), _bcast((8, 64)), _bcast((8, 64)),
                  _bcast((8, 64)), _bcast((8, 64)),
                  _bcast((1, 64)), _bcast((64, 64)), _bcast((1, 64))],
        out_specs=[pl.BlockSpec((BE, 16), lambda i: (i, 0))] * 4,
        out_shape=[jax.ShapeDtypeStruct((n_edges, 16), F32)] * 4,
    )

    node_call = pl.pallas_call(
        _node_body,
        grid=(gn,),
        in_specs=[pl.BlockSpec((BN, 64), lambda i: (i, 0)),
                  pl.BlockSpec((1, BN, 16), lambda i: (0, i, 0)),
                  pl.BlockSpec((1, BN, 16), lambda i: (1, i, 0)),
                  pl.BlockSpec((1, BN, 16), lambda i: (2, i, 0)),
                  pl.BlockSpec((1, BN, 16), lambda i: (3, i, 0)),
                  _bcast((64, 64)), _bcast((64, 64)),
                  _bcast((1, 64)), _bcast((64, 64)), _bcast((1, 64))],
        out_specs=[pl.BlockSpec((BN, 64), lambda i: (i, 0)),
                   pl.BlockSpec((BN, 8), lambda i: (i, 0))],
        out_shape=[jax.ShapeDtypeStruct((n_nodes, 64), F32),
                   jax.ShapeDtypeStruct((n_nodes, 8), F32)],
    )

    for lp in params['layers']:
        we1, be1 = lp['edge1']
        we2, be2 = lp['edge2']
        wn1, bn1 = lp['node1']
        wn2, bn2 = lp['node2']
        z3 = jnp.zeros((3, 64), F32)
        z2 = jnp.zeros((2, 64), F32)
        wi = jnp.concatenate([we1[9:11], we1[0:3], z3], axis=0)
        wj = jnp.concatenate([we1[11:13], we1[3:6], z3], axis=0)
        wp1 = jnp.concatenate([z2, we1[6:7], we1[6:7], we1[6:7], z3], axis=0)
        wp2 = jnp.concatenate([z2, we1[7:8], we1[7:8], we1[7:8], z3], axis=0)
        wea = jnp.concatenate([z2, z3, we1[8:9], z2], axis=0)

        si8, sj8 = gather(s8, dst3_g, src3_g)
        ma, mb, mc, md = edge_call(si8, sj8, ea8, wi, wj, wp1, wp2, wea,
                                   be1.reshape(1, 64), we2,
                                   be2.reshape(1, 64))
        aggr = scatter(ma, mb, mc, md, dst3_s, zeros_tile)
        h, s8 = node_call(h, aggr, aggr, aggr, aggr, wn1[:64], wn1[64:],
                          bn1.reshape(1, 64), wn2, bn2.reshape(1, 64))

    wo1, bo1 = params['out1']
    wo2, bo2 = params['out2']
    out = pl.pallas_call(
        _out_body,
        grid=(gn,),
        in_specs=[pl.BlockSpec((BN, 64), lambda i: (i, 0)),
                  _bcast((64, 64)), _bcast((1, 64)),
                  _bcast((64, 3)), _bcast((1, 3))],
        out_specs=pl.BlockSpec((BN, 3), lambda i: (i, 0)),
        out_shape=jax.ShapeDtypeStruct((n_nodes, 3), F32),
    )(h, wo1, bo1.reshape(1, 64), wo2, bo2.reshape(1, 3))
    return out
